# baseline (speedup 1.0000x reference)
import functools
import jax
import jax.numpy as jnp
from jax import lax
from jax.experimental import pallas as pl
from jax.experimental.pallas import tpu as pltpu
from jax.experimental.pallas import tpu_sc as plsc

B, N_FEAT, D = 16384, 100, 128
NP1 = N_FEAT + 1
NW = 32

B_SC = 4096
B_TC = B - B_SC

ROWS_PER_W = B_SC // NW
G = 2
XC = 16
NCHUNK = ROWS_PER_W // XC
GPC = XC // G
NV = D // 16

_mesh = plsc.VectorSubcoreMesh(core_axis_name="c", subcore_axis_name="s")


@functools.partial(
    pl.kernel,
    mesh=_mesh,
    out_type=jax.ShapeDtypeStruct((B_SC, NP1, D), jnp.float32),
    scratch_types=[
        pltpu.VMEM((NP1, D), jnp.float32),
        pltpu.VMEM((N_FEAT, D), jnp.float32),
        pltpu.VMEM((XC, 128), jnp.float32),
        pltpu.VMEM((G, NP1, D), jnp.float32),
        pltpu.VMEM((G, NP1, D), jnp.float32),
        pltpu.SemaphoreType.DMA,
        pltpu.SemaphoreType.DMA,
    ],
)
def _sc_tok(x_hbm, w_hbm, b_hbm, out_hbm, w_v, b_v, x_v, o_v0, o_v1, sem0, sem1):
    c = lax.axis_index("c")
    s = lax.axis_index("s")
    wid = s * 2 + c
    base = wid * ROWS_PER_W
    pltpu.sync_copy(w_hbm, w_v)
    pltpu.sync_copy(b_hbm, b_v)
    o_bufs = (o_v0, o_v1)
    sems = (sem0, sem1)

    def compute_group(r_local, o_v):
        def feat(n, carry2):
            for j in range(G):
                xs = x_v[r_local + j, pl.ds(n - 1, 16)][0]
                for dv in range(NV):
                    sl = pl.ds(dv * 16, 16)
                    o_v[j, n, sl] = xs * w_v[n, sl] + b_v[n - 1, sl]
            return carry2

        lax.fori_loop(1, NP1, feat, 0)
        for j in range(G):
            for dv in range(NV):
                sl = pl.ds(dv * 16, 16)
                o_v[j, 0, sl] = w_v[0, sl]

    def chunk(ci, carry):
        pltpu.sync_copy(x_hbm.at[pl.ds(base + ci * XC, XC)], x_v)

        def pair(q, carry2):
            for p in range(2):
                gl = q * 2 + p
                row0 = base + ci * XC + gl * G

                @pl.when((ci > 0) | (q > 0))
                def _wait():
                    pltpu.make_async_copy(
                        o_bufs[p], out_hbm.at[pl.ds(row0, G)], sems[p]
                    ).wait()

                compute_group(gl * G, o_bufs[p])
                pltpu.make_async_copy(
                    o_bufs[p], out_hbm.at[pl.ds(row0, G)], sems[p]
                ).start()
            return carry2

        lax.fori_loop(0, GPC // 2, pair, 0)
        return carry

    lax.fori_loop(0, NCHUNK, chunk, 0)
    for p in range(2):
        row0 = base + ROWS_PER_W - (2 - p) * G
        pltpu.make_async_copy(
            o_bufs[p], out_hbm.at[pl.ds(row0, G)], sems[p]
        ).wait()


BB = 64


def _tc_body(xn_ref, w_ref, b_ref, o_ref):
    xn = xn_ref[...]
    o_ref[...] = xn[:, :, None] * w_ref[...][None] + b_ref[...][None]


def _tc_tok(xn, w, bias_p):
    return pl.pallas_call(
        _tc_body,
        grid=(B_TC // BB,),
        in_specs=[
            pl.BlockSpec((BB, NP1), lambda i: (i, 0)),
            pl.BlockSpec((NP1, D), lambda i: (0, 0)),
            pl.BlockSpec((NP1, D), lambda i: (0, 0)),
        ],
        out_specs=pl.BlockSpec((BB, NP1, D), lambda i: (i, 0, 0)),
        out_shape=jax.ShapeDtypeStruct((B_TC, NP1, D), jnp.float32),
    )(xn, w, bias_p)


def kernel(x, numerical_weight, numerical_bias):
    ones = jnp.ones((B_TC, 1), dtype=x.dtype)
    xn_tc = jnp.concatenate([ones, x[:B_TC]], axis=1)
    zero = jnp.zeros((1, D), dtype=numerical_bias.dtype)
    bias_p = jnp.concatenate([zero, numerical_bias], axis=0)
    out_tc = _tc_tok(xn_tc, numerical_weight, bias_p)

    x_sc = jnp.pad(x[B_TC:], ((0, 0), (0, 128 - N_FEAT)))
    out_sc = _sc_tok(x_sc, numerical_weight, numerical_bias)
    return jnp.concatenate([out_tc, out_sc], axis=0)


# speedup vs baseline: 1.8446x; 1.8446x over previous
import jax
import jax.numpy as jnp
from jax import lax
from jax.experimental import pallas as pl
from jax.experimental.pallas import tpu as pltpu

B, N_FEAT, D = 16384, 100, 128
NP1 = N_FEAT + 1
BB = 256
K = 4
SUB = BB // K
NSTEPS = B // BB


def _tok_body(xn_ref, w_ref, b_ref, o_hbm, o_buf, sems):
    i = pl.program_id(0)
    slot = lax.rem(i, 2)

    @pl.when(i >= 2)
    def _wait_prev():
        for k in range(K):
            pltpu.make_async_copy(
                o_buf.at[slot, pl.ds(k * SUB, SUB)],
                o_hbm.at[pl.ds((i - 2) * BB + k * SUB, SUB)],
                sems.at[slot, k],
            ).wait()

    xn = xn_ref[...]
    o_buf[slot] = xn[:, :, None] * w_ref[...][None] + b_ref[...][None]

    for k in range(K):
        pltpu.make_async_copy(
            o_buf.at[slot, pl.ds(k * SUB, SUB)],
            o_hbm.at[pl.ds(i * BB + k * SUB, SUB)],
            sems.at[slot, k],
        ).start()

    @pl.when(i == NSTEPS - 1)
    def _drain():
        for s in range(2):
            base_i = NSTEPS - 2 if s == (NSTEPS - 2) % 2 else NSTEPS - 1
            for k in range(K):
                pltpu.make_async_copy(
                    o_buf.at[s, pl.ds(k * SUB, SUB)],
                    o_hbm.at[pl.ds(base_i * BB + k * SUB, SUB)],
                    sems.at[s, k],
                ).wait()


def kernel(x, numerical_weight, numerical_bias):
    ones = jnp.ones((x.shape[0], 1), dtype=x.dtype)
    xn = jnp.concatenate([ones, x], axis=1)
    zero = jnp.zeros((1, D), dtype=numerical_bias.dtype)
    bias_p = jnp.concatenate([zero, numerical_bias], axis=0)

    return pl.pallas_call(
        _tok_body,
        grid=(NSTEPS,),
        in_specs=[
            pl.BlockSpec((BB, NP1), lambda i: (i, 0)),
            pl.BlockSpec((NP1, D), lambda i: (0, 0)),
            pl.BlockSpec((NP1, D), lambda i: (0, 0)),
        ],
        out_specs=pl.BlockSpec(memory_space=pltpu.MemorySpace.HBM),
        out_shape=jax.ShapeDtypeStruct((B, NP1, D), x.dtype),
        scratch_shapes=[
            pltpu.VMEM((2, BB, NP1, D), jnp.float32),
            pltpu.SemaphoreType.DMA((2, K)),
        ],
        compiler_params=pltpu.CompilerParams(
            dimension_semantics=("arbitrary",),
        ),
    )(xn, numerical_weight, bias_p)
